# baseline (speedup 1.0000x reference)
import jax
import jax.numpy as jnp
from jax import lax
from jax.experimental import pallas as pl
from jax.experimental.pallas import tpu as pltpu
from jax.experimental.pallas import tpu_sc as plsc

NUM_CORES = 2
NUM_SUBCORES = 16
NUM_WORKERS = NUM_CORES * NUM_SUBCORES

NB = 16384
T = 50
TP = 64
D = 32
NUM_EMB = 1000000
PR = NB // 2
WP = PR // NUM_WORKERS
GP = 4
NGROUPS = WP // GP
NBUF = 3


def _emb_kernel(x_hbm, table_hbm, out_hbm, idx_v, rows_v, gsem, osem):
  wid = lax.axis_index("s") * NUM_CORES + lax.axis_index("c")
  base = wid * WP
  pltpu.sync_copy(x_hbm.at[pl.ds(base, WP)], idx_v)

  def fire(g, slot):
    for i in range(GP):
      pltpu.async_copy(
          table_hbm.at[idx_v.at[g * GP + i]],
          rows_v.at[slot, i],
          gsem.at[slot])

  def drain_gathers(slot):
    for i in range(GP):
      pltpu.make_async_copy(
          table_hbm.at[pl.ds(0, 2 * TP)], rows_v.at[slot, i],
          gsem.at[slot]).wait()

  def out_copy(g, slot):
    pltpu.async_copy(
        rows_v.at[slot, pl.ds(0, GP), pl.ds(0, T)],
        out_hbm.at[pl.ds(base + g * GP, GP), 0], osem.at[slot])
    pltpu.async_copy(
        rows_v.at[slot, pl.ds(0, GP), pl.ds(TP, T)],
        out_hbm.at[pl.ds(base + g * GP, GP), 1], osem.at[slot])

  def drain_out(g, slot):
    pltpu.make_async_copy(
        rows_v.at[slot, pl.ds(0, GP), pl.ds(0, T)],
        out_hbm.at[pl.ds(base + g * GP, GP), 0], osem.at[slot]).wait()
    pltpu.make_async_copy(
        rows_v.at[slot, pl.ds(0, GP), pl.ds(TP, T)],
        out_hbm.at[pl.ds(base + g * GP, GP), 1], osem.at[slot]).wait()

  fire(0, 0)
  fire(1, 1)

  def body(g, _):
    slot = g % NBUF

    @pl.when(g + 2 < NGROUPS)
    def _():
      nslot = (g + 2) % NBUF

      @pl.when(g >= 1)
      def _():
        drain_out(g - 1, nslot)
      fire(g + 2, nslot)

    drain_gathers(slot)
    out_copy(g, slot)
    return 0

  lax.fori_loop(0, NGROUPS, body, 0)
  drain_out(NGROUPS - 3, (NGROUPS - 3) % NBUF)
  drain_out(NGROUPS - 2, (NGROUPS - 2) % NBUF)
  drain_out(NGROUPS - 1, (NGROUPS - 1) % NBUF)


@jax.jit
def _emb(x_pairs, table):
  mesh = plsc.VectorSubcoreMesh(
      core_axis_name="c", subcore_axis_name="s",
      num_cores=NUM_CORES, num_subcores=NUM_SUBCORES)
  f = pl.kernel(
      _emb_kernel,
      out_type=jax.ShapeDtypeStruct((PR, 2, T, D), jnp.float32),
      mesh=mesh,
      scratch_types=[
          pltpu.VMEM((WP, 2 * TP), jnp.int32),
          pltpu.VMEM((NBUF, GP, 2 * TP, D), jnp.float32),
          pltpu.SemaphoreType.DMA((NBUF,)),
          pltpu.SemaphoreType.DMA((NBUF,)),
      ],
      compiler_params=pltpu.CompilerParams(
          use_tc_tiling_on_sc=False, skip_device_barrier=True),
  )
  return f(x_pairs, table)


def kernel(x, table):
  pad = (jnp.arange(NB, dtype=jnp.int32)[:, None] * (TP - T)
         + jnp.arange(TP - T, dtype=jnp.int32)[None, :]) % NUM_EMB
  x_pairs = jnp.concatenate([x.astype(jnp.int32), pad], axis=1).reshape(
      PR, 2 * TP)
  out = _emb(x_pairs, table)
  return out.reshape(NB, T, D)


# speedup vs baseline: 1.0066x; 1.0066x over previous
import jax
import jax.numpy as jnp
from jax import lax
from jax.experimental import pallas as pl
from jax.experimental.pallas import tpu as pltpu
from jax.experimental.pallas import tpu_sc as plsc

NUM_CORES = 2
NUM_SUBCORES = 16
NUM_WORKERS = NUM_CORES * NUM_SUBCORES

NB = 16384
T = 50
TP = 56
D = 32
NUM_EMB = 1000000
WB = NB // NUM_WORKERS
GB = 16
NGROUPS = WB // GB
NBUF = 3


def _emb_kernel(x_hbm, table_hbm, out_hbm, idx_v, rows_v, gsem, osem):
  wid = lax.axis_index("s") * NUM_CORES + lax.axis_index("c")
  base = wid * WB
  pltpu.sync_copy(x_hbm.at[pl.ds(base, WB)], idx_v)

  def fire(g, slot):
    for i in range(GB):
      pltpu.async_copy(
          table_hbm.at[idx_v.at[g * GB + i]],
          rows_v.at[slot, i],
          gsem.at[slot])

  def drain_gathers(slot):
    for i in range(GB):
      pltpu.make_async_copy(
          table_hbm.at[pl.ds(0, TP)], rows_v.at[slot, i],
          gsem.at[slot]).wait()

  def out_copy(g, slot):
    pltpu.async_copy(
        rows_v.at[slot, pl.ds(0, GB), pl.ds(0, T)],
        out_hbm.at[pl.ds(base + g * GB, GB)], osem.at[slot])

  def drain_out(g, slot):
    pltpu.make_async_copy(
        rows_v.at[slot, pl.ds(0, GB), pl.ds(0, T)],
        out_hbm.at[pl.ds(base + g * GB, GB)],
        osem.at[slot]).wait()

  fire(0, 0)
  fire(1, 1)

  def body(g, _):
    slot = g % NBUF

    @pl.when(g + 2 < NGROUPS)
    def _():
      nslot = (g + 2) % NBUF

      @pl.when(g >= 1)
      def _():
        drain_out(g - 1, nslot)
      fire(g + 2, nslot)

    drain_gathers(slot)
    out_copy(g, slot)
    return 0

  lax.fori_loop(0, NGROUPS, body, 0)
  drain_out(NGROUPS - 3, (NGROUPS - 3) % NBUF)
  drain_out(NGROUPS - 2, (NGROUPS - 2) % NBUF)
  drain_out(NGROUPS - 1, (NGROUPS - 1) % NBUF)


@jax.jit
def _emb(x_pad, table):
  mesh = plsc.VectorSubcoreMesh(
      core_axis_name="c", subcore_axis_name="s",
      num_cores=NUM_CORES, num_subcores=NUM_SUBCORES)
  f = pl.kernel(
      _emb_kernel,
      out_type=jax.ShapeDtypeStruct((NB, T, D), jnp.float32),
      mesh=mesh,
      scratch_types=[
          pltpu.VMEM((WB, TP), jnp.int32),
          pltpu.VMEM((NBUF, GB, TP, D), jnp.float32),
          pltpu.SemaphoreType.DMA((NBUF,)),
          pltpu.SemaphoreType.DMA((NBUF,)),
      ],
      compiler_params=pltpu.CompilerParams(
          use_tc_tiling_on_sc=False, skip_device_barrier=True),
  )
  return f(x_pad, table)


def kernel(x, table):
  pad = (jnp.arange(NB, dtype=jnp.int32)[:, None] * (TP - T)
         + jnp.arange(TP - T, dtype=jnp.int32)[None, :]) % NUM_EMB
  x_pad = jnp.concatenate([x.astype(jnp.int32), pad], axis=1)
  return _emb(x_pad, table)
